# Initial kernel scaffold; baseline (speedup 1.0000x reference)
import jax
import jax.numpy as jnp
from jax.experimental import pallas as pl


def kernel(t, h, edge_index, W1, b1, W2, b2):
    raise NotImplementedError("write your pallas kernel here")


# speedup vs baseline: 4.8517x; 4.8517x over previous
import functools

import jax
import jax.numpy as jnp
from jax import lax
from jax.experimental import pallas as pl
from jax.experimental.pallas import tpu as pltpu
from jax.experimental.pallas import tpu_sc as plsc

N = 10000
E = 320000
D = 128
NP = 10240

NC = 2
NS = 16
NW = NC * NS
EPW = E // NW
CH = 80
NCHUNK = EPW // CH

ROWS_PER_TILE = NP // NS
ZR = 128


def _sc_mesh():
    return plsc.VectorSubcoreMesh(core_axis_name="c", subcore_axis_name="s")


def _deg_body(src_hbm, dst_hbm, out_hbm, idx_v, ones_v, zb_v, sdo, sdi):
    cid = lax.axis_index("c")
    sid = lax.axis_index("s")
    wid = cid * NS + sid

    for j in range(CH // 16):
        ones_v[pl.ds(16 * j, 16)] = jnp.ones((16,), jnp.float32)

    def _z(i, _):
        zb_v[pl.ds(16 * i, 16)] = jnp.zeros((16,), jnp.float32)
        return 0
    lax.fori_loop(0, (NP // NS) // 16, _z, 0)

    seg = NP // NS
    pltpu.sync_copy(zb_v, sdo.at[pl.ds(sid * seg, seg)])
    pltpu.sync_copy(zb_v, sdi.at[pl.ds(sid * seg, seg)])
    plsc.subcore_barrier()

    def _count(c, _):
        base = wid * EPW + c * CH
        pltpu.sync_copy(src_hbm.at[pl.ds(base, CH)], idx_v)
        pltpu.sync_copy(ones_v, sdo.at[idx_v], add=True)
        pltpu.sync_copy(dst_hbm.at[pl.ds(base, CH)], idx_v)
        pltpu.sync_copy(ones_v, sdi.at[idx_v], add=True)
        return 0
    lax.fori_loop(0, NCHUNK, _count, 0)

    plsc.subcore_barrier()
    pltpu.sync_copy(sdo.at[pl.ds(sid * seg, seg)],
                    out_hbm.at[cid, 0, pl.ds(sid * seg, seg)])
    pltpu.sync_copy(sdi.at[pl.ds(sid * seg, seg)],
                    out_hbm.at[cid, 1, pl.ds(sid * seg, seg)])


def _sc_degrees(src, dst):
    k = pl.kernel(
        _deg_body,
        out_type=jax.ShapeDtypeStruct((NC, 2, NP), jnp.float32),
        mesh=_sc_mesh(),
        scratch_types=[
            pltpu.VMEM((CH,), jnp.int32),
            pltpu.VMEM((CH,), jnp.float32),
            pltpu.VMEM((NP // NS,), jnp.float32),
            pltpu.VMEM_SHARED((NP,), jnp.float32),
            pltpu.VMEM_SHARED((NP,), jnp.float32),
        ],
    )
    return k(src, dst)


def _agg_body(hw_hbm, src_hbm, dst_hbm, out_hbm, idx_s, idx_d, rows, zrow, sagg, sem):
    cid = lax.axis_index("c")
    sid = lax.axis_index("s")
    wid = cid * NS + sid

    def _z(r, _):
        for j in range(D // 16):
            zrow[r, pl.ds(16 * j, 16)] = jnp.zeros((16,), jnp.float32)
        return 0
    lax.fori_loop(0, ZR, _z, 0)
    for kk in range(ROWS_PER_TILE // ZR):
        pltpu.sync_copy(zrow, sagg.at[pl.ds(sid * ROWS_PER_TILE + kk * ZR, ZR), :])
    plsc.subcore_barrier()

    def _edge(c, _):
        base = wid * EPW + c * CH
        pltpu.sync_copy(src_hbm.at[pl.ds(base, CH)], idx_s)
        pltpu.async_copy(hw_hbm.at[idx_s], rows, sem).wait()
        pltpu.sync_copy(dst_hbm.at[pl.ds(base, CH)], idx_d)
        pltpu.sync_copy(rows, sagg.at[idx_d], add=True)
        return 0
    lax.fori_loop(0, NCHUNK, _edge, 0)

    plsc.subcore_barrier()
    for kk in range(ROWS_PER_TILE // ZR):
        r0 = sid * ROWS_PER_TILE + kk * ZR
        pltpu.sync_copy(sagg.at[pl.ds(r0, ZR), :], out_hbm.at[cid, pl.ds(r0, ZR), :])


def _sc_aggregate(hw, src, dst):
    k = pl.kernel(
        _agg_body,
        out_type=jax.ShapeDtypeStruct((NC, NP, D), jnp.float32),
        mesh=_sc_mesh(),
        scratch_types=[
            pltpu.VMEM((CH,), jnp.int32),
            pltpu.VMEM((CH,), jnp.int32),
            pltpu.VMEM((CH, D), jnp.float32),
            pltpu.VMEM((ZR, D), jnp.float32),
            pltpu.VMEM_SHARED((NP, D), jnp.float32),
            pltpu.SemaphoreType.DMA,
        ],
    )
    return k(hw, src, dst)


def _norm_body(degp_ref, out_ref):
    deg = degp_ref[0] + degp_ref[1]
    out_ref[...] = lax.rsqrt(jnp.maximum(deg, 1.0))


def _tc_norms(degp):
    return pl.pallas_call(
        _norm_body,
        out_shape=jax.ShapeDtypeStruct((2, NP), jnp.float32),
    )(degp)


_RB = 1000


def _mm_body(x_ref, w_ref, no_ref, out_ref):
    y = jnp.dot(x_ref[...], w_ref[...], preferred_element_type=jnp.float32)
    out_ref[...] = y * no_ref[...]


def _tc_matmul_scale(x, w, no):
    return pl.pallas_call(
        _mm_body,
        grid=(N // _RB,),
        in_specs=[
            pl.BlockSpec((_RB, D), lambda i: (i, 0)),
            pl.BlockSpec((D, D), lambda i: (0, 0)),
            pl.BlockSpec((_RB, 1), lambda i: (i, 0)),
        ],
        out_specs=pl.BlockSpec((_RB, D), lambda i: (i, 0)),
        out_shape=jax.ShapeDtypeStruct((N, D), jnp.float32),
    )(x, w, no)


def _fuse_body(aggp_ref, ni_ref, b_ref, w_ref, no_ref, out_ref):
    x = (aggp_ref[0] + aggp_ref[1]) * ni_ref[...] + b_ref[...]
    x = jnp.maximum(x, 0.0)
    y = jnp.dot(x, w_ref[...], preferred_element_type=jnp.float32)
    out_ref[...] = y * no_ref[...]


def _tc_fuse(aggp, ni, b, w, no):
    return pl.pallas_call(
        _fuse_body,
        grid=(N // _RB,),
        in_specs=[
            pl.BlockSpec((2, _RB, D), lambda i: (0, i, 0)),
            pl.BlockSpec((_RB, 1), lambda i: (i, 0)),
            pl.BlockSpec((1, D), lambda i: (0, 0)),
            pl.BlockSpec((D, D), lambda i: (0, 0)),
            pl.BlockSpec((_RB, 1), lambda i: (i, 0)),
        ],
        out_specs=pl.BlockSpec((_RB, D), lambda i: (i, 0)),
        out_shape=jax.ShapeDtypeStruct((N, D), jnp.float32),
    )(aggp, ni, b, w, no)


def _final_body(aggp_ref, ni_ref, b_ref, out_ref):
    out_ref[...] = (aggp_ref[0] + aggp_ref[1]) * ni_ref[...] + b_ref[...]


def _tc_final(aggp, ni, b):
    return pl.pallas_call(
        _final_body,
        grid=(N // _RB,),
        in_specs=[
            pl.BlockSpec((2, _RB, D), lambda i: (0, i, 0)),
            pl.BlockSpec((_RB, 1), lambda i: (i, 0)),
            pl.BlockSpec((1, D), lambda i: (0, 0)),
        ],
        out_specs=pl.BlockSpec((_RB, D), lambda i: (i, 0)),
        out_shape=jax.ShapeDtypeStruct((N, D), jnp.float32),
    )(aggp, ni, b)


def kernel(t, h, edge_index, W1, b1, W2, b2):
    src = edge_index[0]
    dst = edge_index[1]

    degp = _sc_degrees(src, dst)
    norms = _tc_norms(degp)
    no = norms[0, :N].reshape(N, 1)
    ni = norms[1, :N].reshape(N, 1)
    b1r = b1.reshape(1, D)
    b2r = b2.reshape(1, D)

    hw1 = _tc_matmul_scale(h, W1, no)
    agg1 = _sc_aggregate(hw1, src, dst)[:, :N, :]
    hw2 = _tc_fuse(agg1, ni, b1r, W2, no)
    agg2 = _sc_aggregate(hw2, src, dst)[:, :N, :]
    return _tc_final(agg2, ni, b2r)


# speedup vs baseline: 12.1220x; 2.4985x over previous
import jax
import jax.numpy as jnp
from jax import lax
from jax.experimental import pallas as pl
from jax.experimental.pallas import tpu as pltpu
from jax.experimental.pallas import tpu_sc as plsc

N = 10000
E = 320000
D = 128
NP = 10240

NC = 2
NS = 16
NW = NC * NS
EPW = E // NW

CHD = 80
NCHD = EPW // CHD
NBD = 5
NGD = NCHD // NBD

CH = 40
EPWP = 10240
EPAD = NW * EPWP
NCHA = EPWP // CH
SPC = 8
NSPAN = NCHA // SPC
NB = 4

ROWS_PER_TILE = NP // NS
ZR = 128


def _sc_mesh():
    return plsc.VectorSubcoreMesh(core_axis_name="c", subcore_axis_name="s")


def _deg_body(src_hbm, dst_hbm, out_hbm, srcv, dstv, ones_v, zb_v, sdo, sdi,
              sem_a, sem_b):
    cid = lax.axis_index("c")
    sid = lax.axis_index("s")
    wid = cid * NS + sid

    for j in range(CHD // 16):
        ones_v[pl.ds(16 * j, 16)] = jnp.ones((16,), jnp.float32)

    def _z(i, _):
        zb_v[pl.ds(16 * i, 16)] = jnp.zeros((16,), jnp.float32)
        return 0
    lax.fori_loop(0, (NP // NS) // 16, _z, 0)

    seg = NP // NS
    pltpu.sync_copy(zb_v, sdo.at[pl.ds(sid * seg, seg)])
    pltpu.sync_copy(zb_v, sdi.at[pl.ds(sid * seg, seg)])
    pltpu.sync_copy(src_hbm.at[wid], srcv)
    pltpu.sync_copy(dst_hbm.at[wid], dstv)
    plsc.subcore_barrier()

    def _count(g, _):
        for b in range(NBD):
            row = g * NBD + b
            pltpu.async_copy(ones_v, sdo.at[srcv.at[row]], sem_a, add=True)
            pltpu.async_copy(ones_v, sdi.at[dstv.at[row]], sem_b, add=True)
        for b in range(NBD):
            row = g * NBD + b
            pltpu.make_async_copy(ones_v, sdo.at[srcv.at[row]], sem_a).wait()
            pltpu.make_async_copy(ones_v, sdi.at[dstv.at[row]], sem_b).wait()
        return 0
    lax.fori_loop(0, NGD, _count, 0)

    plsc.subcore_barrier()
    pltpu.sync_copy(sdo.at[pl.ds(sid * seg, seg)],
                    out_hbm.at[cid, 0, pl.ds(sid * seg, seg)])
    pltpu.sync_copy(sdi.at[pl.ds(sid * seg, seg)],
                    out_hbm.at[cid, 1, pl.ds(sid * seg, seg)])


def _sc_degrees(src3, dst3):
    k = pl.kernel(
        _deg_body,
        out_type=jax.ShapeDtypeStruct((NC, 2, NP), jnp.float32),
        mesh=_sc_mesh(),
        scratch_types=[
            pltpu.VMEM((NCHD, CHD), jnp.int32),
            pltpu.VMEM((NCHD, CHD), jnp.int32),
            pltpu.VMEM((CHD,), jnp.float32),
            pltpu.VMEM((NP // NS,), jnp.float32),
            pltpu.VMEM_SHARED((NP,), jnp.float32),
            pltpu.VMEM_SHARED((NP,), jnp.float32),
            pltpu.SemaphoreType.DMA,
            pltpu.SemaphoreType.DMA,
        ],
    )
    return k(src3, dst3)


def _agg_body(hw_hbm, srcf_hbm, dst3_hbm, out_hbm, sagg, srcv, dstv,
              r0, r1, r2, r3, g0, g1, g2, g3, s0, s1, s2, s3, isem):
    rows = (r0, r1, r2, r3)
    gsem = (g0, g1, g2, g3)
    ssem = (s0, s1, s2, s3)
    cid = lax.axis_index("c")
    sid = lax.axis_index("s")
    wid = cid * NS + sid

    def _z(r, _):
        for j in range(D // 16):
            rows[0][r, pl.ds(16 * j, 16)] = jnp.zeros((16,), jnp.float32)
        return 0
    lax.fori_loop(0, CH, _z, 0)
    for kk in range(ROWS_PER_TILE // CH):
        pltpu.sync_copy(rows[0],
                        sagg.at[pl.ds(sid * ROWS_PER_TILE + kk * CH, CH), :])
    pltpu.sync_copy(srcf_hbm.at[pl.ds(wid * EPWP, EPWP)], srcv)
    pltpu.sync_copy(dst3_hbm.at[wid, pl.ds(0, SPC), :], dstv.at[0])
    plsc.subcore_barrier()

    def _span(s, _):
        p = lax.rem(s, 2)

        @pl.when(s + 1 < NSPAN)
        def _prefetch():
            pltpu.async_copy(dst3_hbm.at[wid, pl.ds((s + 1) * SPC, SPC), :],
                             dstv.at[1 - p], isem)

        for half in range(SPC // NB):
            for b in range(NB):
                ch = half * NB + b
                gidx = srcv.at[pl.ds((s * SPC + ch) * CH, CH)]
                if half == 0:
                    @pl.when(s > 0)
                    def _wait_prev(b=b):
                        pltpu.make_async_copy(
                            rows[b], sagg.at[dstv.at[p, 0]], ssem[b]).wait()
                else:
                    pltpu.make_async_copy(
                        rows[b], sagg.at[dstv.at[p, 0]], ssem[b]).wait()
                pltpu.async_copy(hw_hbm.at[gidx], rows[b], gsem[b])
            for b in range(NB):
                ch = half * NB + b
                gidx = srcv.at[pl.ds((s * SPC + ch) * CH, CH)]
                pltpu.make_async_copy(hw_hbm.at[gidx], rows[b], gsem[b]).wait()
                pltpu.async_copy(rows[b], sagg.at[dstv.at[p, ch]], ssem[b],
                                 add=True)

        @pl.when(s + 1 < NSPAN)
        def _wait_prefetch():
            pltpu.make_async_copy(dst3_hbm.at[wid, pl.ds((s + 1) * SPC, SPC), :],
                                  dstv.at[1 - p], isem).wait()
        return 0
    lax.fori_loop(0, NSPAN, _span, 0)

    for b in range(NB):
        pltpu.make_async_copy(rows[b], sagg.at[dstv.at[0, 0]], ssem[b]).wait()
    plsc.subcore_barrier()
    for kk in range(ROWS_PER_TILE // ZR):
        r0_ = sid * ROWS_PER_TILE + kk * ZR
        pltpu.sync_copy(sagg.at[pl.ds(r0_, ZR), :],
                        out_hbm.at[cid, pl.ds(r0_, ZR), :])


def _sc_aggregate(hw, srcf, dst3):
    k = pl.kernel(
        _agg_body,
        out_type=jax.ShapeDtypeStruct((NC, NP, D), jnp.float32),
        mesh=_sc_mesh(),
        scratch_types=[
            pltpu.VMEM_SHARED((NP, D), jnp.float32),
            pltpu.VMEM((EPWP,), jnp.int32),
            pltpu.VMEM((2, SPC, CH), jnp.int32),
        ] + [pltpu.VMEM((CH, D), jnp.float32)] * NB
          + [pltpu.SemaphoreType.DMA] * (2 * NB + 1),
    )
    return k(hw, srcf, dst3)


def _norm_body(degp_ref, out_ref):
    deg = degp_ref[0] + degp_ref[1]
    out_ref[...] = lax.rsqrt(jnp.maximum(deg, 1.0))


def _tc_norms(degp):
    return pl.pallas_call(
        _norm_body,
        out_shape=jax.ShapeDtypeStruct((2, NP), jnp.float32),
    )(degp)


_RB = 1000
_RBP = 1024


def _mm_body(x_ref, w_ref, no_ref, out_ref):
    y = jnp.dot(x_ref[...], w_ref[...], preferred_element_type=jnp.float32)
    out_ref[...] = y * no_ref[...]


def _tc_matmul_scale(x, w, no):
    return pl.pallas_call(
        _mm_body,
        grid=(N // _RB,),
        in_specs=[
            pl.BlockSpec((_RB, D), lambda i: (i, 0)),
            pl.BlockSpec((D, D), lambda i: (0, 0)),
            pl.BlockSpec((_RB, 1), lambda i: (i, 0)),
        ],
        out_specs=pl.BlockSpec((_RB, D), lambda i: (i, 0)),
        out_shape=jax.ShapeDtypeStruct((N, D), jnp.float32),
    )(x, w, no)


def _fuse_body(aggp_ref, ni_ref, b_ref, w_ref, no_ref, out_ref):
    x = (aggp_ref[0] + aggp_ref[1]) * ni_ref[...] + b_ref[...]
    x = jnp.maximum(x, 0.0)
    y = jnp.dot(x, w_ref[...], preferred_element_type=jnp.float32)
    out_ref[...] = y * no_ref[...]


def _tc_fuse(aggp, ni, b, w, no):
    return pl.pallas_call(
        _fuse_body,
        grid=(NP // _RBP,),
        in_specs=[
            pl.BlockSpec((2, _RBP, D), lambda i: (0, i, 0)),
            pl.BlockSpec((_RBP, 1), lambda i: (i, 0)),
            pl.BlockSpec((1, D), lambda i: (0, 0)),
            pl.BlockSpec((D, D), lambda i: (0, 0)),
            pl.BlockSpec((_RBP, 1), lambda i: (i, 0)),
        ],
        out_specs=pl.BlockSpec((_RBP, D), lambda i: (i, 0)),
        out_shape=jax.ShapeDtypeStruct((NP, D), jnp.float32),
    )(aggp, ni, b, w, no)


def _final_body(aggp_ref, ni_ref, b_ref, out_ref):
    out_ref[...] = (aggp_ref[0] + aggp_ref[1]) * ni_ref[...] + b_ref[...]


def _tc_final(aggp, ni, b):
    return pl.pallas_call(
        _final_body,
        grid=(NP // _RBP,),
        in_specs=[
            pl.BlockSpec((2, _RBP, D), lambda i: (0, i, 0)),
            pl.BlockSpec((_RBP, 1), lambda i: (i, 0)),
            pl.BlockSpec((1, D), lambda i: (0, 0)),
        ],
        out_specs=pl.BlockSpec((_RBP, D), lambda i: (i, 0)),
        out_shape=jax.ShapeDtypeStruct((NP, D), jnp.float32),
    )(aggp, ni, b)


def kernel(t, h, edge_index, W1, b1, W2, b2):
    src3 = edge_index[0].reshape(NW, NCHD, CHD)
    dst3 = edge_index[1].reshape(NW, NCHD, CHD)
    npad = EPAD - E
    pad_src = jnp.arange(npad, dtype=jnp.int32) % N
    pad_dst = N + (jnp.arange(npad, dtype=jnp.int32) % (NP - N))
    srcf = jnp.concatenate([edge_index[0], pad_src])
    dst3a = jnp.concatenate([edge_index[1], pad_dst]).reshape(NW, NCHA, CH)

    degp = _sc_degrees(src3, dst3)
    norms = _tc_norms(degp)
    no_p = norms[0].reshape(NP, 1)
    ni_p = norms[1].reshape(NP, 1)
    no_n = no_p[:N]
    b1r = b1.reshape(1, D)
    b2r = b2.reshape(1, D)

    hw1 = _tc_matmul_scale(h, W1, no_n)
    agg1 = _sc_aggregate(hw1, srcf, dst3a)
    hw2 = _tc_fuse(agg1, ni_p, b1r, W2, no_p)
    agg2 = _sc_aggregate(hw2, srcf, dst3a)
    out = _tc_final(agg2, ni_p, b2r)
    return out[:N]


# speedup vs baseline: 12.5819x; 1.0379x over previous
import jax
import jax.numpy as jnp
from jax import lax
from jax.experimental import pallas as pl
from jax.experimental.pallas import tpu as pltpu
from jax.experimental.pallas import tpu_sc as plsc

N = 10000
E = 320000
D = 128
NP = 10240

NC = 2
NS = 16
NW = NC * NS
EPW = E // NW

CHD = 80
NCHD = EPW // CHD
NBD = 5
NGD = NCHD // NBD

CH = 64
EPWP = 10240
EPAD = NW * EPWP
NCHA = EPWP // CH
SPC = 8
NSPAN = NCHA // SPC
NB = 4

ROWS_PER_TILE = NP // NS
ZR = 128


def _sc_mesh():
    return plsc.VectorSubcoreMesh(core_axis_name="c", subcore_axis_name="s")


def _deg_body(src_hbm, dst_hbm, out_hbm, srcv, dstv, ones_v, zb_v, sdo, sdi,
              sem_a, sem_b):
    cid = lax.axis_index("c")
    sid = lax.axis_index("s")
    wid = cid * NS + sid

    for j in range(CHD // 16):
        ones_v[pl.ds(16 * j, 16)] = jnp.ones((16,), jnp.float32)

    def _z(i, _):
        zb_v[pl.ds(16 * i, 16)] = jnp.zeros((16,), jnp.float32)
        return 0
    lax.fori_loop(0, (NP // NS) // 16, _z, 0)

    seg = NP // NS
    pltpu.sync_copy(zb_v, sdo.at[pl.ds(sid * seg, seg)])
    pltpu.sync_copy(zb_v, sdi.at[pl.ds(sid * seg, seg)])
    pltpu.sync_copy(src_hbm.at[wid], srcv)
    pltpu.sync_copy(dst_hbm.at[wid], dstv)
    plsc.subcore_barrier()

    def _count(g, _):
        for b in range(NBD):
            row = g * NBD + b
            pltpu.async_copy(ones_v, sdo.at[srcv.at[row]], sem_a, add=True)
            pltpu.async_copy(ones_v, sdi.at[dstv.at[row]], sem_b, add=True)
        for b in range(NBD):
            row = g * NBD + b
            pltpu.make_async_copy(ones_v, sdo.at[srcv.at[row]], sem_a).wait()
            pltpu.make_async_copy(ones_v, sdi.at[dstv.at[row]], sem_b).wait()
        return 0
    lax.fori_loop(0, NGD, _count, 0)

    plsc.subcore_barrier()
    pltpu.sync_copy(sdo.at[pl.ds(sid * seg, seg)],
                    out_hbm.at[cid, 0, pl.ds(sid * seg, seg)])
    pltpu.sync_copy(sdi.at[pl.ds(sid * seg, seg)],
                    out_hbm.at[cid, 1, pl.ds(sid * seg, seg)])


def _sc_degrees(src3, dst3):
    k = pl.kernel(
        _deg_body,
        out_type=jax.ShapeDtypeStruct((NC, 2, NP), jnp.float32),
        mesh=_sc_mesh(),
        scratch_types=[
            pltpu.VMEM((NCHD, CHD), jnp.int32),
            pltpu.VMEM((NCHD, CHD), jnp.int32),
            pltpu.VMEM((CHD,), jnp.float32),
            pltpu.VMEM((NP // NS,), jnp.float32),
            pltpu.VMEM_SHARED((NP,), jnp.float32),
            pltpu.VMEM_SHARED((NP,), jnp.float32),
            pltpu.SemaphoreType.DMA,
            pltpu.SemaphoreType.DMA,
        ],
    )
    return k(src3, dst3)


def _agg_body(hw_hbm, src3_hbm, dst3_hbm, out_hbm, sagg, srcv, dstv,
              r0, r1, r2, r3, g0, g1, g2, g3, s0, s1, s2, s3, isem_s, isem_d):
    rows = (r0, r1, r2, r3)
    gsem = (g0, g1, g2, g3)
    ssem = (s0, s1, s2, s3)
    cid = lax.axis_index("c")
    sid = lax.axis_index("s")
    wid = cid * NS + sid

    def _z(r, _):
        for j in range(D // 16):
            rows[0][r, pl.ds(16 * j, 16)] = jnp.zeros((16,), jnp.float32)
        return 0
    lax.fori_loop(0, CH, _z, 0)
    for kk in range(ROWS_PER_TILE // CH):
        pltpu.sync_copy(rows[0],
                        sagg.at[pl.ds(sid * ROWS_PER_TILE + kk * CH, CH), :])
    pltpu.sync_copy(src3_hbm.at[wid, pl.ds(0, SPC), :], srcv.at[0])
    pltpu.sync_copy(dst3_hbm.at[wid, pl.ds(0, SPC), :], dstv.at[0])
    plsc.subcore_barrier()

    def _span(s, _):
        p = lax.rem(s, 2)

        @pl.when(s + 1 < NSPAN)
        def _prefetch():
            pltpu.async_copy(src3_hbm.at[wid, pl.ds((s + 1) * SPC, SPC), :],
                             srcv.at[1 - p], isem_s)
            pltpu.async_copy(dst3_hbm.at[wid, pl.ds((s + 1) * SPC, SPC), :],
                             dstv.at[1 - p], isem_d)

        for half in range(SPC // NB):
            for b in range(NB):
                ch = half * NB + b
                if half == 0:
                    @pl.when(s > 0)
                    def _wait_prev(b=b):
                        pltpu.make_async_copy(
                            rows[b], sagg.at[dstv.at[p, 0]], ssem[b]).wait()
                else:
                    pltpu.make_async_copy(
                        rows[b], sagg.at[dstv.at[p, 0]], ssem[b]).wait()
                pltpu.async_copy(hw_hbm.at[srcv.at[p, ch]], rows[b], gsem[b])
            for b in range(NB):
                ch = half * NB + b
                pltpu.make_async_copy(
                    hw_hbm.at[srcv.at[p, ch]], rows[b], gsem[b]).wait()
                pltpu.async_copy(rows[b], sagg.at[dstv.at[p, ch]], ssem[b],
                                 add=True)

        @pl.when(s + 1 < NSPAN)
        def _wait_prefetch():
            pltpu.make_async_copy(src3_hbm.at[wid, pl.ds((s + 1) * SPC, SPC), :],
                                  srcv.at[1 - p], isem_s).wait()
            pltpu.make_async_copy(dst3_hbm.at[wid, pl.ds((s + 1) * SPC, SPC), :],
                                  dstv.at[1 - p], isem_d).wait()
        return 0
    lax.fori_loop(0, NSPAN, _span, 0)

    for b in range(NB):
        pltpu.make_async_copy(rows[b], sagg.at[dstv.at[0, 0]], ssem[b]).wait()
    plsc.subcore_barrier()
    for kk in range(ROWS_PER_TILE // ZR):
        r0_ = sid * ROWS_PER_TILE + kk * ZR
        pltpu.sync_copy(sagg.at[pl.ds(r0_, ZR), :],
                        out_hbm.at[cid, pl.ds(r0_, ZR), :])


def _sc_aggregate(hw, src3, dst3):
    k = pl.kernel(
        _agg_body,
        out_type=jax.ShapeDtypeStruct((NC, NP, D), jnp.float32),
        mesh=_sc_mesh(),
        scratch_types=[
            pltpu.VMEM_SHARED((NP, D), jnp.float32),
            pltpu.VMEM((2, SPC, CH), jnp.int32),
            pltpu.VMEM((2, SPC, CH), jnp.int32),
        ] + [pltpu.VMEM((CH, D), jnp.float32)] * NB
          + [pltpu.SemaphoreType.DMA] * (2 * NB + 2),
    )
    return k(hw, src3, dst3)


def _norm_body(degp_ref, out_ref):
    deg = degp_ref[0] + degp_ref[1]
    out_ref[...] = lax.rsqrt(jnp.maximum(deg, 1.0))


def _tc_norms(degp):
    return pl.pallas_call(
        _norm_body,
        out_shape=jax.ShapeDtypeStruct((2, NP), jnp.float32),
    )(degp)


_RB = 1000
_RBP = 1024


def _mm_body(x_ref, w_ref, no_ref, out_ref):
    y = jnp.dot(x_ref[...], w_ref[...], preferred_element_type=jnp.float32)
    out_ref[...] = y * no_ref[...]


def _tc_matmul_scale(x, w, no):
    return pl.pallas_call(
        _mm_body,
        grid=(N // _RB,),
        in_specs=[
            pl.BlockSpec((_RB, D), lambda i: (i, 0)),
            pl.BlockSpec((D, D), lambda i: (0, 0)),
            pl.BlockSpec((_RB, 1), lambda i: (i, 0)),
        ],
        out_specs=pl.BlockSpec((_RB, D), lambda i: (i, 0)),
        out_shape=jax.ShapeDtypeStruct((N, D), jnp.float32),
    )(x, w, no)


def _fuse_body(aggp_ref, ni_ref, b_ref, w_ref, no_ref, out_ref):
    x = (aggp_ref[0] + aggp_ref[1]) * ni_ref[...] + b_ref[...]
    x = jnp.maximum(x, 0.0)
    y = jnp.dot(x, w_ref[...], preferred_element_type=jnp.float32)
    out_ref[...] = y * no_ref[...]


def _tc_fuse(aggp, ni, b, w, no):
    return pl.pallas_call(
        _fuse_body,
        grid=(NP // _RBP,),
        in_specs=[
            pl.BlockSpec((2, _RBP, D), lambda i: (0, i, 0)),
            pl.BlockSpec((_RBP, 1), lambda i: (i, 0)),
            pl.BlockSpec((1, D), lambda i: (0, 0)),
            pl.BlockSpec((D, D), lambda i: (0, 0)),
            pl.BlockSpec((_RBP, 1), lambda i: (i, 0)),
        ],
        out_specs=pl.BlockSpec((_RBP, D), lambda i: (i, 0)),
        out_shape=jax.ShapeDtypeStruct((NP, D), jnp.float32),
    )(aggp, ni, b, w, no)


def _final_body(aggp_ref, ni_ref, b_ref, out_ref):
    out_ref[...] = (aggp_ref[0] + aggp_ref[1]) * ni_ref[...] + b_ref[...]


def _tc_final(aggp, ni, b):
    return pl.pallas_call(
        _final_body,
        grid=(NP // _RBP,),
        in_specs=[
            pl.BlockSpec((2, _RBP, D), lambda i: (0, i, 0)),
            pl.BlockSpec((_RBP, 1), lambda i: (i, 0)),
            pl.BlockSpec((1, D), lambda i: (0, 0)),
        ],
        out_specs=pl.BlockSpec((_RBP, D), lambda i: (i, 0)),
        out_shape=jax.ShapeDtypeStruct((NP, D), jnp.float32),
    )(aggp, ni, b)


def kernel(t, h, edge_index, W1, b1, W2, b2):
    src3 = edge_index[0].reshape(NW, NCHD, CHD)
    dst3 = edge_index[1].reshape(NW, NCHD, CHD)
    npad = EPAD - E
    pad_src = jnp.arange(npad, dtype=jnp.int32) % N
    pad_dst = N + (jnp.arange(npad, dtype=jnp.int32) % (NP - N))
    src3a = jnp.concatenate([edge_index[0], pad_src]).reshape(NW, NCHA, CH)
    dst3a = jnp.concatenate([edge_index[1], pad_dst]).reshape(NW, NCHA, CH)

    degp = _sc_degrees(src3, dst3)
    norms = _tc_norms(degp)
    no_p = norms[0].reshape(NP, 1)
    ni_p = norms[1].reshape(NP, 1)
    no_n = no_p[:N]
    b1r = b1.reshape(1, D)
    b2r = b2.reshape(1, D)

    hw1 = _tc_matmul_scale(h, W1, no_n)
    agg1 = _sc_aggregate(hw1, src3a, dst3a)
    hw2 = _tc_fuse(agg1, ni_p, b1r, W2, no_p)
    agg2 = _sc_aggregate(hw2, src3a, dst3a)
    out = _tc_final(agg2, ni_p, b2r)
    return out[:N]


# speedup vs baseline: 12.9836x; 1.0319x over previous
import jax
import jax.numpy as jnp
from jax import lax
from jax.experimental import pallas as pl
from jax.experimental.pallas import tpu as pltpu
from jax.experimental.pallas import tpu_sc as plsc

N = 10000
E = 320000
D = 128
NP = 10240

NC = 2
NS = 16
NW = NC * NS
EPW = E // NW

CHD = 80
NCHD = (E // NS) // CHD
NBD = 5
NGD = NCHD // NBD

CH = 64
EPWP = 10240
EPAD = NW * EPWP
NCHA = EPWP // CH
SPC = 8
NSPAN = NCHA // SPC
NB = 4

ROWS_PER_TILE = NP // NS
ZR = 128


def _sc_mesh():
    return plsc.VectorSubcoreMesh(core_axis_name="c", subcore_axis_name="s")


def _rsqrt_nr(d):
    one = jnp.full((16,), 1.0, jnp.float32)
    c15 = jnp.full((16,), 1.5, jnp.float32)
    ch = jnp.full((16,), 0.5, jnp.float32)
    x = one / d
    for _ in range(20):
        x = x * (c15 - ch * d * x * x)
    return x


def _deg_body(edges4_hbm, out_hbm, idxv, ones_v, zb_v, nv, sdeg, sem_a):
    cid = lax.axis_index("c")
    sid = lax.axis_index("s")

    for j in range(CHD // 16):
        ones_v[pl.ds(16 * j, 16)] = jnp.ones((16,), jnp.float32)

    def _z(i, _):
        zb_v[pl.ds(16 * i, 16)] = jnp.zeros((16,), jnp.float32)
        return 0
    lax.fori_loop(0, (NP // NS) // 16, _z, 0)

    seg = NP // NS
    pltpu.sync_copy(zb_v, sdeg.at[pl.ds(sid * seg, seg)])
    pltpu.sync_copy(edges4_hbm.at[cid, sid], idxv)
    plsc.subcore_barrier()

    def _count(g, _):
        for b in range(NBD):
            row = g * NBD + b
            pltpu.async_copy(ones_v, sdeg.at[idxv.at[row]], sem_a, add=True)
        for b in range(NBD):
            row = g * NBD + b
            pltpu.make_async_copy(ones_v, sdeg.at[idxv.at[row]], sem_a).wait()
        return 0
    lax.fori_loop(0, NGD, _count, 0)

    plsc.subcore_barrier()
    pltpu.sync_copy(sdeg.at[pl.ds(sid * seg, seg)], zb_v)
    def _n(i, _):
        d = jnp.maximum(zb_v[pl.ds(16 * i, 16)], jnp.full((16,), 1.0, jnp.float32))
        nv[pl.ds(16 * i, 16)] = _rsqrt_nr(d)
        return 0
    lax.fori_loop(0, seg // 16, _n, 0)
    pltpu.sync_copy(nv, out_hbm.at[cid, pl.ds(sid * seg, seg)])


def _sc_degrees(edges4):
    k = pl.kernel(
        _deg_body,
        out_type=jax.ShapeDtypeStruct((NC, NP), jnp.float32),
        mesh=_sc_mesh(),
        scratch_types=[
            pltpu.VMEM((NCHD, CHD), jnp.int32),
            pltpu.VMEM((CHD,), jnp.float32),
            pltpu.VMEM((NP // NS,), jnp.float32),
            pltpu.VMEM((NP // NS,), jnp.float32),
            pltpu.VMEM_SHARED((NP,), jnp.float32),
            pltpu.SemaphoreType.DMA,
        ],
    )
    return k(edges4)


def _agg_body(hw_hbm, src3_hbm, dst3_hbm, out_hbm, sagg, srcv, dstv,
              r0, r1, r2, r3, g0, g1, g2, g3, s0, s1, s2, s3, isem_s, isem_d):
    rows = (r0, r1, r2, r3)
    gsem = (g0, g1, g2, g3)
    ssem = (s0, s1, s2, s3)
    cid = lax.axis_index("c")
    sid = lax.axis_index("s")
    wid = cid * NS + sid

    def _z(r, _):
        for j in range(D // 16):
            rows[0][r, pl.ds(16 * j, 16)] = jnp.zeros((16,), jnp.float32)
        return 0
    lax.fori_loop(0, CH, _z, 0)
    for kk in range(ROWS_PER_TILE // CH):
        pltpu.sync_copy(rows[0],
                        sagg.at[pl.ds(sid * ROWS_PER_TILE + kk * CH, CH), :])
    pltpu.sync_copy(src3_hbm.at[wid, pl.ds(0, SPC), :], srcv.at[0])
    pltpu.sync_copy(dst3_hbm.at[wid, pl.ds(0, SPC), :], dstv.at[0])
    plsc.subcore_barrier()

    def _span(s, _):
        p = lax.rem(s, 2)

        @pl.when(s + 1 < NSPAN)
        def _prefetch():
            pltpu.async_copy(src3_hbm.at[wid, pl.ds((s + 1) * SPC, SPC), :],
                             srcv.at[1 - p], isem_s)
            pltpu.async_copy(dst3_hbm.at[wid, pl.ds((s + 1) * SPC, SPC), :],
                             dstv.at[1 - p], isem_d)

        for half in range(SPC // NB):
            for b in range(NB):
                ch = half * NB + b
                if half == 0:
                    @pl.when(s > 0)
                    def _wait_prev(b=b):
                        pltpu.make_async_copy(
                            rows[b], sagg.at[dstv.at[p, 0]], ssem[b]).wait()
                else:
                    pltpu.make_async_copy(
                        rows[b], sagg.at[dstv.at[p, 0]], ssem[b]).wait()
                pltpu.async_copy(hw_hbm.at[srcv.at[p, ch]], rows[b], gsem[b])
            for b in range(NB):
                ch = half * NB + b
                pltpu.make_async_copy(
                    hw_hbm.at[srcv.at[p, ch]], rows[b], gsem[b]).wait()
                pltpu.async_copy(rows[b], sagg.at[dstv.at[p, ch]], ssem[b],
                                 add=True)

        @pl.when(s + 1 < NSPAN)
        def _wait_prefetch():
            pltpu.make_async_copy(src3_hbm.at[wid, pl.ds((s + 1) * SPC, SPC), :],
                                  srcv.at[1 - p], isem_s).wait()
            pltpu.make_async_copy(dst3_hbm.at[wid, pl.ds((s + 1) * SPC, SPC), :],
                                  dstv.at[1 - p], isem_d).wait()
        return 0
    lax.fori_loop(0, NSPAN, _span, 0)

    for b in range(NB):
        pltpu.make_async_copy(rows[b], sagg.at[dstv.at[0, 0]], ssem[b]).wait()
    plsc.subcore_barrier()
    for kk in range(ROWS_PER_TILE // ZR):
        r0_ = sid * ROWS_PER_TILE + kk * ZR
        pltpu.sync_copy(sagg.at[pl.ds(r0_, ZR), :],
                        out_hbm.at[cid, pl.ds(r0_, ZR), :])


def _sc_aggregate(hw, src3, dst3):
    k = pl.kernel(
        _agg_body,
        out_type=jax.ShapeDtypeStruct((NC, NP, D), jnp.float32),
        mesh=_sc_mesh(),
        scratch_types=[
            pltpu.VMEM_SHARED((NP, D), jnp.float32),
            pltpu.VMEM((2, SPC, CH), jnp.int32),
            pltpu.VMEM((2, SPC, CH), jnp.int32),
        ] + [pltpu.VMEM((CH, D), jnp.float32)] * NB
          + [pltpu.SemaphoreType.DMA] * (2 * NB + 2),
    )
    return k(hw, src3, dst3)


_RB = 1000
_RBP = 1024


def _mm_body(x_ref, w_ref, no_ref, out_ref):
    y = jnp.dot(x_ref[...], w_ref[...], preferred_element_type=jnp.float32)
    out_ref[...] = y * no_ref[...]


def _tc_matmul_scale(x, w, no):
    return pl.pallas_call(
        _mm_body,
        grid=(N // _RB,),
        in_specs=[
            pl.BlockSpec((_RB, D), lambda i: (i, 0)),
            pl.BlockSpec((D, D), lambda i: (0, 0)),
            pl.BlockSpec((_RB, 1), lambda i: (i, 0)),
        ],
        out_specs=pl.BlockSpec((_RB, D), lambda i: (i, 0)),
        out_shape=jax.ShapeDtypeStruct((N, D), jnp.float32),
    )(x, w, no)


def _fuse_body(aggp_ref, ni_ref, b_ref, w_ref, no_ref, out_ref):
    x = (aggp_ref[0] + aggp_ref[1]) * ni_ref[...] + b_ref[...]
    x = jnp.maximum(x, 0.0)
    y = jnp.dot(x, w_ref[...], preferred_element_type=jnp.float32)
    out_ref[...] = y * no_ref[...]


def _tc_fuse(aggp, ni, b, w, no):
    return pl.pallas_call(
        _fuse_body,
        grid=(NP // _RBP,),
        in_specs=[
            pl.BlockSpec((2, _RBP, D), lambda i: (0, i, 0)),
            pl.BlockSpec((_RBP, 1), lambda i: (i, 0)),
            pl.BlockSpec((1, D), lambda i: (0, 0)),
            pl.BlockSpec((D, D), lambda i: (0, 0)),
            pl.BlockSpec((_RBP, 1), lambda i: (i, 0)),
        ],
        out_specs=pl.BlockSpec((_RBP, D), lambda i: (i, 0)),
        out_shape=jax.ShapeDtypeStruct((NP, D), jnp.float32),
    )(aggp, ni, b, w, no)


def _final_body(aggp_ref, ni_ref, b_ref, out_ref):
    out_ref[...] = (aggp_ref[0] + aggp_ref[1]) * ni_ref[...] + b_ref[...]


def _tc_final(aggp, ni, b):
    return pl.pallas_call(
        _final_body,
        grid=(NP // _RBP,),
        in_specs=[
            pl.BlockSpec((2, _RBP, D), lambda i: (0, i, 0)),
            pl.BlockSpec((_RBP, 1), lambda i: (i, 0)),
            pl.BlockSpec((1, D), lambda i: (0, 0)),
        ],
        out_specs=pl.BlockSpec((_RBP, D), lambda i: (i, 0)),
        out_shape=jax.ShapeDtypeStruct((NP, D), jnp.float32),
    )(aggp, ni, b)


def kernel(t, h, edge_index, W1, b1, W2, b2):
    edges4 = edge_index.reshape(2, NS, NCHD, CHD)
    npad = EPAD - E
    pad_src = jnp.arange(npad, dtype=jnp.int32) % N
    pad_dst = N + (jnp.arange(npad, dtype=jnp.int32) % (NP - N))
    src3a = jnp.concatenate([edge_index[0], pad_src]).reshape(NW, NCHA, CH)
    dst3a = jnp.concatenate([edge_index[1], pad_dst]).reshape(NW, NCHA, CH)

    norms = _sc_degrees(edges4)
    no_p = norms[0].reshape(NP, 1)
    ni_p = norms[1].reshape(NP, 1)
    no_n = no_p[:N]
    b1r = b1.reshape(1, D)
    b2r = b2.reshape(1, D)

    hw1 = _tc_matmul_scale(h, W1, no_n)
    agg1 = _sc_aggregate(hw1, src3a, dst3a)
    hw2 = _tc_fuse(agg1, ni_p, b1r, W2, no_p)
    agg2 = _sc_aggregate(hw2, src3a, dst3a)
    out = _tc_final(agg2, ni_p, b2r)
    return out[:N]


# speedup vs baseline: 13.0687x; 1.0066x over previous
import jax
import jax.numpy as jnp
from jax import lax
from jax.experimental import pallas as pl
from jax.experimental.pallas import tpu as pltpu
from jax.experimental.pallas import tpu_sc as plsc

N = 10000
E = 320000
D = 128
NP = 10240

NC = 2
NS = 16
NW = NC * NS
EPW = E // NW

CHD = 80
NCHD = (E // NS) // CHD
NBD = 5
NGD = NCHD // NBD

CH = 64
EPWP = 10240
EPAD = NW * EPWP
NCHA = EPWP // CH
SPC = 8
NSPAN = NCHA // SPC
NB = 4

ROWS_PER_TILE = NP // NS
ZR = 128


def _sc_mesh():
    return plsc.VectorSubcoreMesh(core_axis_name="c", subcore_axis_name="s")


def _rsqrt_nr(d):
    one = jnp.full((16,), 1.0, jnp.float32)
    c15 = jnp.full((16,), 1.5, jnp.float32)
    ch = jnp.full((16,), 0.5, jnp.float32)
    x = one / d
    for _ in range(20):
        x = x * (c15 - ch * d * x * x)
    return x


def _deg_body(edges4_hbm, out_hbm, idxv, ones_v, zb_v, nv, sdeg, sem_a):
    cid = lax.axis_index("c")
    sid = lax.axis_index("s")

    for j in range(CHD // 16):
        ones_v[pl.ds(16 * j, 16)] = jnp.ones((16,), jnp.float32)

    def _z(i, _):
        zb_v[pl.ds(16 * i, 16)] = jnp.zeros((16,), jnp.float32)
        return 0
    lax.fori_loop(0, (NP // NS) // 16, _z, 0)

    seg = NP // NS
    pltpu.sync_copy(zb_v, sdeg.at[pl.ds(sid * seg, seg)])
    pltpu.sync_copy(edges4_hbm.at[cid, sid], idxv)
    plsc.subcore_barrier()

    def _count(g, _):
        for b in range(NBD):
            row = g * NBD + b
            pltpu.async_copy(ones_v, sdeg.at[idxv.at[row]], sem_a, add=True)
        for b in range(NBD):
            row = g * NBD + b
            pltpu.make_async_copy(ones_v, sdeg.at[idxv.at[row]], sem_a).wait()
        return 0
    lax.fori_loop(0, NGD, _count, 0)

    plsc.subcore_barrier()
    pltpu.sync_copy(sdeg.at[pl.ds(sid * seg, seg)], zb_v)
    def _n(i, _):
        d = jnp.maximum(zb_v[pl.ds(16 * i, 16)], jnp.full((16,), 1.0, jnp.float32))
        nv[pl.ds(16 * i, 16)] = _rsqrt_nr(d)
        return 0
    lax.fori_loop(0, seg // 16, _n, 0)
    pltpu.sync_copy(nv, out_hbm.at[cid, pl.ds(sid * seg, seg)])


def _sc_degrees(edges4):
    k = pl.kernel(
        _deg_body,
        out_type=jax.ShapeDtypeStruct((NC, NP), jnp.float32),
        mesh=_sc_mesh(),
        scratch_types=[
            pltpu.VMEM((NCHD, CHD), jnp.int32),
            pltpu.VMEM((CHD,), jnp.float32),
            pltpu.VMEM((NP // NS,), jnp.float32),
            pltpu.VMEM((NP // NS,), jnp.float32),
            pltpu.VMEM_SHARED((NP,), jnp.float32),
            pltpu.SemaphoreType.DMA,
        ],
    )
    return k(edges4)


def _agg_body(hw_hbm, src3_hbm, dst3_hbm, out_hbm, sagg, srcv, dstv,
              r0, r1, r2, r3, g0, g1, g2, g3, s0, s1, s2, s3, isem_s, isem_d):
    rows = (r0, r1, r2, r3)
    gsem = (g0, g1, g2, g3)
    ssem = (s0, s1, s2, s3)
    cid = lax.axis_index("c")
    sid = lax.axis_index("s")
    wid = cid * NS + sid

    def _z(r, _):
        for j in range(D // 16):
            rows[0][r, pl.ds(16 * j, 16)] = jnp.zeros((16,), jnp.float32)
        return 0
    lax.fori_loop(0, CH, _z, 0)
    pltpu.async_copy(src3_hbm.at[wid, pl.ds(0, SPC), :], srcv.at[0], isem_s)
    pltpu.async_copy(dst3_hbm.at[wid, pl.ds(0, SPC), :], dstv.at[0], isem_d)
    for kk in range(ROWS_PER_TILE // CH):
        pltpu.async_copy(rows[0],
                         sagg.at[pl.ds(sid * ROWS_PER_TILE + kk * CH, CH), :],
                         gsem[0])
    for kk in range(ROWS_PER_TILE // CH):
        pltpu.make_async_copy(
            rows[0], sagg.at[pl.ds(sid * ROWS_PER_TILE + kk * CH, CH), :],
            gsem[0]).wait()
    pltpu.make_async_copy(src3_hbm.at[wid, pl.ds(0, SPC), :], srcv.at[0],
                          isem_s).wait()
    pltpu.make_async_copy(dst3_hbm.at[wid, pl.ds(0, SPC), :], dstv.at[0],
                          isem_d).wait()
    plsc.subcore_barrier()

    def _span(s, _):
        p = lax.rem(s, 2)

        @pl.when(s + 1 < NSPAN)
        def _prefetch():
            pltpu.async_copy(src3_hbm.at[wid, pl.ds((s + 1) * SPC, SPC), :],
                             srcv.at[1 - p], isem_s)
            pltpu.async_copy(dst3_hbm.at[wid, pl.ds((s + 1) * SPC, SPC), :],
                             dstv.at[1 - p], isem_d)

        for half in range(SPC // NB):
            for b in range(NB):
                ch = half * NB + b
                if half == 0:
                    @pl.when(s > 0)
                    def _wait_prev(b=b):
                        pltpu.make_async_copy(
                            rows[b], sagg.at[dstv.at[p, 0]], ssem[b]).wait()
                else:
                    pltpu.make_async_copy(
                        rows[b], sagg.at[dstv.at[p, 0]], ssem[b]).wait()
                pltpu.async_copy(hw_hbm.at[srcv.at[p, ch]], rows[b], gsem[b])
            for b in range(NB):
                ch = half * NB + b
                pltpu.make_async_copy(
                    hw_hbm.at[srcv.at[p, ch]], rows[b], gsem[b]).wait()
                pltpu.async_copy(rows[b], sagg.at[dstv.at[p, ch]], ssem[b],
                                 add=True)

        @pl.when(s + 1 < NSPAN)
        def _wait_prefetch():
            pltpu.make_async_copy(src3_hbm.at[wid, pl.ds((s + 1) * SPC, SPC), :],
                                  srcv.at[1 - p], isem_s).wait()
            pltpu.make_async_copy(dst3_hbm.at[wid, pl.ds((s + 1) * SPC, SPC), :],
                                  dstv.at[1 - p], isem_d).wait()
        return 0
    lax.fori_loop(0, NSPAN, _span, 0)

    for b in range(NB):
        pltpu.make_async_copy(rows[b], sagg.at[dstv.at[0, 0]], ssem[b]).wait()
    plsc.subcore_barrier()
    for kk in range(ROWS_PER_TILE // ZR):
        r0_ = sid * ROWS_PER_TILE + kk * ZR
        pltpu.async_copy(sagg.at[pl.ds(r0_, ZR), :],
                         out_hbm.at[cid, pl.ds(r0_, ZR), :], gsem[kk % NB])
    for kk in range(ROWS_PER_TILE // ZR):
        r0_ = sid * ROWS_PER_TILE + kk * ZR
        pltpu.make_async_copy(sagg.at[pl.ds(r0_, ZR), :],
                              out_hbm.at[cid, pl.ds(r0_, ZR), :],
                              gsem[kk % NB]).wait()


def _sc_aggregate(hw, src3, dst3):
    k = pl.kernel(
        _agg_body,
        out_type=jax.ShapeDtypeStruct((NC, NP, D), jnp.float32),
        mesh=_sc_mesh(),
        scratch_types=[
            pltpu.VMEM_SHARED((NP, D), jnp.float32),
            pltpu.VMEM((2, SPC, CH), jnp.int32),
            pltpu.VMEM((2, SPC, CH), jnp.int32),
        ] + [pltpu.VMEM((CH, D), jnp.float32)] * NB
          + [pltpu.SemaphoreType.DMA] * (2 * NB + 2),
    )
    return k(hw, src3, dst3)


_RB = 1000
_RBP = 1024


def _mm_body(x_ref, w_ref, no_ref, out_ref):
    y = jnp.dot(x_ref[...], w_ref[...], preferred_element_type=jnp.float32)
    out_ref[...] = y * no_ref[...]


def _tc_matmul_scale(x, w, no):
    return pl.pallas_call(
        _mm_body,
        grid=(N // _RB,),
        in_specs=[
            pl.BlockSpec((_RB, D), lambda i: (i, 0)),
            pl.BlockSpec((D, D), lambda i: (0, 0)),
            pl.BlockSpec((_RB, 1), lambda i: (i, 0)),
        ],
        out_specs=pl.BlockSpec((_RB, D), lambda i: (i, 0)),
        out_shape=jax.ShapeDtypeStruct((N, D), jnp.float32),
    )(x, w, no)


def _fuse_body(aggp_ref, ni_ref, b_ref, w_ref, no_ref, out_ref):
    x = (aggp_ref[0] + aggp_ref[1]) * ni_ref[...] + b_ref[...]
    x = jnp.maximum(x, 0.0)
    y = jnp.dot(x, w_ref[...], preferred_element_type=jnp.float32)
    out_ref[...] = y * no_ref[...]


def _tc_fuse(aggp, ni, b, w, no):
    return pl.pallas_call(
        _fuse_body,
        grid=(NP // _RBP,),
        in_specs=[
            pl.BlockSpec((2, _RBP, D), lambda i: (0, i, 0)),
            pl.BlockSpec((_RBP, 1), lambda i: (i, 0)),
            pl.BlockSpec((1, D), lambda i: (0, 0)),
            pl.BlockSpec((D, D), lambda i: (0, 0)),
            pl.BlockSpec((_RBP, 1), lambda i: (i, 0)),
        ],
        out_specs=pl.BlockSpec((_RBP, D), lambda i: (i, 0)),
        out_shape=jax.ShapeDtypeStruct((NP, D), jnp.float32),
    )(aggp, ni, b, w, no)


def _final_body(aggp_ref, ni_ref, b_ref, out_ref):
    out_ref[...] = (aggp_ref[0] + aggp_ref[1]) * ni_ref[...] + b_ref[...]


def _tc_final(aggp, ni, b):
    return pl.pallas_call(
        _final_body,
        grid=(NP // _RBP,),
        in_specs=[
            pl.BlockSpec((2, _RBP, D), lambda i: (0, i, 0)),
            pl.BlockSpec((_RBP, 1), lambda i: (i, 0)),
            pl.BlockSpec((1, D), lambda i: (0, 0)),
        ],
        out_specs=pl.BlockSpec((_RBP, D), lambda i: (i, 0)),
        out_shape=jax.ShapeDtypeStruct((NP, D), jnp.float32),
    )(aggp, ni, b)


def kernel(t, h, edge_index, W1, b1, W2, b2):
    edges4 = edge_index.reshape(2, NS, NCHD, CHD)
    npad = EPAD - E
    pad_src = jnp.arange(npad, dtype=jnp.int32) % N
    pad_dst = N + (jnp.arange(npad, dtype=jnp.int32) % (NP - N))
    src3a = jnp.concatenate([edge_index[0], pad_src]).reshape(NW, NCHA, CH)
    dst3a = jnp.concatenate([edge_index[1], pad_dst]).reshape(NW, NCHA, CH)

    norms = _sc_degrees(edges4)
    no_p = norms[0].reshape(NP, 1)
    ni_p = norms[1].reshape(NP, 1)
    no_n = no_p[:N]
    b1r = b1.reshape(1, D)
    b2r = b2.reshape(1, D)

    hw1 = _tc_matmul_scale(h, W1, no_n)
    agg1 = _sc_aggregate(hw1, src3a, dst3a)
    hw2 = _tc_fuse(agg1, ni_p, b1r, W2, no_p)
    agg2 = _sc_aggregate(hw2, src3a, dst3a)
    out = _tc_final(agg2, ni_p, b2r)
    return out[:N]


# speedup vs baseline: 13.2842x; 1.0165x over previous
import jax
import jax.numpy as jnp
from jax import lax
from jax.experimental import pallas as pl
from jax.experimental.pallas import tpu as pltpu
from jax.experimental.pallas import tpu_sc as plsc

N = 10000
E = 320000
D = 128
NP = 10240

NC = 2
NS = 16
NW = NC * NS
EPW = E // NW

CHD = 80
NCHD = (E // NS) // CHD
NBD = 5
NGD = NCHD // NBD

CH = 64
EPWP = 10240
EPAD = NW * EPWP
NCHA = EPWP // CH
SPC = 8
NSPAN = NCHA // SPC
NB = 4

ROWS_PER_TILE = NP // NS
ZR = 128


def _sc_mesh():
    return plsc.VectorSubcoreMesh(core_axis_name="c", subcore_axis_name="s")


def _rsqrt_nr(d):
    one = jnp.full((16,), 1.0, jnp.float32)
    c15 = jnp.full((16,), 1.5, jnp.float32)
    ch = jnp.full((16,), 0.5, jnp.float32)
    x = one / d
    for _ in range(20):
        x = x * (c15 - ch * d * x * x)
    return x


def _deg_body(edges4_hbm, out_hbm, idxv, ones_v, zb_v, nv, sdeg, sem_a):
    cid = lax.axis_index("c")
    sid = lax.axis_index("s")

    for j in range(CHD // 16):
        ones_v[pl.ds(16 * j, 16)] = jnp.ones((16,), jnp.float32)

    def _z(i, _):
        zb_v[pl.ds(16 * i, 16)] = jnp.zeros((16,), jnp.float32)
        return 0
    lax.fori_loop(0, (NP // NS) // 16, _z, 0)

    seg = NP // NS
    pltpu.sync_copy(zb_v, sdeg.at[pl.ds(sid * seg, seg)])
    pltpu.sync_copy(edges4_hbm.at[cid, sid], idxv)
    plsc.subcore_barrier()

    def _count(g, _):
        for b in range(NBD):
            row = g * NBD + b
            pltpu.async_copy(ones_v, sdeg.at[idxv.at[row]], sem_a, add=True)
        for b in range(NBD):
            row = g * NBD + b
            pltpu.make_async_copy(ones_v, sdeg.at[idxv.at[row]], sem_a).wait()
        return 0
    lax.fori_loop(0, NGD, _count, 0)

    plsc.subcore_barrier()
    pltpu.sync_copy(sdeg.at[pl.ds(sid * seg, seg)], zb_v)
    def _n(i, _):
        d = jnp.maximum(zb_v[pl.ds(16 * i, 16)], jnp.full((16,), 1.0, jnp.float32))
        nv[pl.ds(16 * i, 16)] = _rsqrt_nr(d)
        return 0
    lax.fori_loop(0, seg // 16, _n, 0)
    pltpu.sync_copy(nv, out_hbm.at[cid, pl.ds(sid * seg, seg)])


def _sc_degrees(edges4):
    k = pl.kernel(
        _deg_body,
        out_type=jax.ShapeDtypeStruct((NC, NP), jnp.float32),
        mesh=_sc_mesh(),
        scratch_types=[
            pltpu.VMEM((NCHD, CHD), jnp.int32),
            pltpu.VMEM((CHD,), jnp.float32),
            pltpu.VMEM((NP // NS,), jnp.float32),
            pltpu.VMEM((NP // NS,), jnp.float32),
            pltpu.VMEM_SHARED((NP,), jnp.float32),
            pltpu.SemaphoreType.DMA,
        ],
    )
    return k(edges4)


def _agg_body(hw_hbm, src3_hbm, dst3_hbm, out_hbm, sagg, srcv, dstv,
              r0, r1, r2, r3, g0, g1, g2, g3, s0, s1, s2, s3, isem_s, isem_d):
    rows = (r0, r1, r2, r3)
    gsem = (g0, g1, g2, g3)
    ssem = (s0, s1, s2, s3)
    cid = lax.axis_index("c")
    sid = lax.axis_index("s")
    wid = cid * NS + sid

    def _z(r, _):
        for j in range(D // 16):
            rows[0][r, pl.ds(16 * j, 16)] = jnp.zeros((16,), jnp.float32)
        return 0
    lax.fori_loop(0, CH, _z, 0)
    pltpu.async_copy(src3_hbm.at[wid, pl.ds(0, SPC), :], srcv.at[0], isem_s)
    pltpu.async_copy(dst3_hbm.at[wid, pl.ds(0, SPC), :], dstv.at[0], isem_d)
    for kk in range(ROWS_PER_TILE // CH):
        pltpu.async_copy(rows[0],
                         sagg.at[pl.ds(sid * ROWS_PER_TILE + kk * CH, CH), :],
                         gsem[0])
    for kk in range(ROWS_PER_TILE // CH):
        pltpu.make_async_copy(
            rows[0], sagg.at[pl.ds(sid * ROWS_PER_TILE + kk * CH, CH), :],
            gsem[0]).wait()
    pltpu.make_async_copy(src3_hbm.at[wid, pl.ds(0, SPC), :], srcv.at[0],
                          isem_s).wait()
    pltpu.make_async_copy(dst3_hbm.at[wid, pl.ds(0, SPC), :], dstv.at[0],
                          isem_d).wait()
    plsc.subcore_barrier()

    def _span(s, _):
        p = lax.rem(s, 2)

        @pl.when(s + 1 < NSPAN)
        def _prefetch():
            pltpu.async_copy(src3_hbm.at[wid, pl.ds((s + 1) * SPC, SPC), :],
                             srcv.at[1 - p], isem_s)
            pltpu.async_copy(dst3_hbm.at[wid, pl.ds((s + 1) * SPC, SPC), :],
                             dstv.at[1 - p], isem_d)

        for half in range(SPC // NB):
            for b in range(NB):
                ch = half * NB + b
                if half == 0:
                    @pl.when(s > 0)
                    def _wait_prev(b=b):
                        pltpu.make_async_copy(
                            rows[b], sagg.at[dstv.at[p, 0]], ssem[b]).wait()
                else:
                    pltpu.make_async_copy(
                        rows[b], sagg.at[dstv.at[p, 0]], ssem[b]).wait()
                pltpu.async_copy(hw_hbm.at[srcv.at[p, ch]], rows[b], gsem[b])
            for b in range(NB):
                ch = half * NB + b
                pltpu.make_async_copy(
                    hw_hbm.at[srcv.at[p, ch]], rows[b], gsem[b]).wait()
                pltpu.async_copy(rows[b], sagg.at[dstv.at[p, ch]], ssem[b],
                                 add=True)

        @pl.when(s + 1 < NSPAN)
        def _wait_prefetch():
            pltpu.make_async_copy(src3_hbm.at[wid, pl.ds((s + 1) * SPC, SPC), :],
                                  srcv.at[1 - p], isem_s).wait()
            pltpu.make_async_copy(dst3_hbm.at[wid, pl.ds((s + 1) * SPC, SPC), :],
                                  dstv.at[1 - p], isem_d).wait()
        return 0
    lax.fori_loop(0, NSPAN, _span, 0)

    for b in range(NB):
        pltpu.make_async_copy(rows[b], sagg.at[dstv.at[0, 0]], ssem[b]).wait()
    plsc.subcore_barrier()
    for kk in range(ROWS_PER_TILE // ZR):
        r0_ = sid * ROWS_PER_TILE + kk * ZR
        pltpu.async_copy(sagg.at[pl.ds(r0_, ZR), :],
                         out_hbm.at[cid, pl.ds(r0_, ZR), :], gsem[kk % NB])
    for kk in range(ROWS_PER_TILE // ZR):
        r0_ = sid * ROWS_PER_TILE + kk * ZR
        pltpu.make_async_copy(sagg.at[pl.ds(r0_, ZR), :],
                              out_hbm.at[cid, pl.ds(r0_, ZR), :],
                              gsem[kk % NB]).wait()


def _sc_aggregate(hw, src3, dst3):
    k = pl.kernel(
        _agg_body,
        out_type=jax.ShapeDtypeStruct((NC, NP, D), jnp.float32),
        mesh=_sc_mesh(),
        scratch_types=[
            pltpu.VMEM_SHARED((NP, D), jnp.float32),
            pltpu.VMEM((2, SPC, CH), jnp.int32),
            pltpu.VMEM((2, SPC, CH), jnp.int32),
        ] + [pltpu.VMEM((CH, D), jnp.float32)] * NB
          + [pltpu.SemaphoreType.DMA] * (2 * NB + 2),
    )
    return k(hw, src3, dst3)


_RB = 1000
_RBP = 1024


def _mm_body(x_ref, w_ref, no_ref, out_ref):
    y = jnp.dot(x_ref[...], w_ref[...], preferred_element_type=jnp.float32)
    out_ref[...] = y * no_ref[...]


def _tc_matmul_scale(x, w, no):
    return pl.pallas_call(
        _mm_body,
        grid=(N // _RB,),
        in_specs=[
            pl.BlockSpec((_RB, D), lambda i: (i, 0)),
            pl.BlockSpec((D, D), lambda i: (0, 0)),
            pl.BlockSpec((_RB, 1), lambda i: (i, 0)),
        ],
        out_specs=pl.BlockSpec((_RB, D), lambda i: (i, 0)),
        out_shape=jax.ShapeDtypeStruct((N, D), jnp.float32),
    )(x, w, no)


def _fuse_body(aggp_ref, ni_ref, b_ref, w_ref, no_ref, out_ref):
    x = (aggp_ref[0] + aggp_ref[1]) * ni_ref[...] + b_ref[...]
    x = jnp.maximum(x, 0.0)
    y = jnp.dot(x, w_ref[...], preferred_element_type=jnp.float32)
    out_ref[...] = y * no_ref[...]


def _tc_fuse(aggp, ni, b, w, no):
    return pl.pallas_call(
        _fuse_body,
        grid=(N // _RB,),
        in_specs=[
            pl.BlockSpec((2, _RB, D), lambda i: (0, i, 0)),
            pl.BlockSpec((_RB, 1), lambda i: (i, 0)),
            pl.BlockSpec((1, D), lambda i: (0, 0)),
            pl.BlockSpec((D, D), lambda i: (0, 0)),
            pl.BlockSpec((_RB, 1), lambda i: (i, 0)),
        ],
        out_specs=pl.BlockSpec((_RB, D), lambda i: (i, 0)),
        out_shape=jax.ShapeDtypeStruct((N, D), jnp.float32),
    )(aggp, ni, b, w, no)


def _final_body(aggp_ref, ni_ref, b_ref, out_ref):
    out_ref[...] = (aggp_ref[0] + aggp_ref[1]) * ni_ref[...] + b_ref[...]


def _tc_final(aggp, ni, b):
    return pl.pallas_call(
        _final_body,
        grid=(N // _RB,),
        in_specs=[
            pl.BlockSpec((2, _RB, D), lambda i: (0, i, 0)),
            pl.BlockSpec((_RB, 1), lambda i: (i, 0)),
            pl.BlockSpec((1, D), lambda i: (0, 0)),
        ],
        out_specs=pl.BlockSpec((_RB, D), lambda i: (i, 0)),
        out_shape=jax.ShapeDtypeStruct((N, D), jnp.float32),
    )(aggp, ni, b)


def kernel(t, h, edge_index, W1, b1, W2, b2):
    edges4 = edge_index.reshape(2, NS, NCHD, CHD)
    npad = EPAD - E
    pad_src = jnp.arange(npad, dtype=jnp.int32) % N
    pad_dst = N + (jnp.arange(npad, dtype=jnp.int32) % (NP - N))
    src3a = jnp.concatenate([edge_index[0], pad_src]).reshape(NW, NCHA, CH)
    dst3a = jnp.concatenate([edge_index[1], pad_dst]).reshape(NW, NCHA, CH)

    norms = _sc_degrees(edges4)
    no_p = norms[0].reshape(NP, 1)
    ni_p = norms[1].reshape(NP, 1)
    no_n = no_p[:N]
    ni_n = ni_p[:N]
    b1r = b1.reshape(1, D)
    b2r = b2.reshape(1, D)

    hw1 = _tc_matmul_scale(h, W1, no_n)
    agg1 = _sc_aggregate(hw1, src3a, dst3a)
    hw2 = _tc_fuse(agg1, ni_n, b1r, W2, no_n)
    agg2 = _sc_aggregate(hw2, src3a, dst3a)
    return _tc_final(agg2, ni_n, b2r)
